# Initial kernel scaffold; baseline (speedup 1.0000x reference)
import jax
import jax.numpy as jnp
from jax.experimental import pallas as pl


def kernel(node_in, edge_in, senders, receivers, params):
    raise NotImplementedError("write your pallas kernel here")


# speedup vs baseline: 16.1452x; 16.1452x over previous
import functools

import jax
import jax.numpy as jnp
from jax import lax
from jax.experimental import pallas as pl
from jax.experimental.pallas import tpu as pltpu
from jax.experimental.pallas import tpu_sc as plsc

N_NODES = 50000
N_EDGES = 800000
D = 64
STEPS = 8

PN = 51200
PE = 802816
GW = 128
NODE_BLK = 2048
EDGE_BLK = 4096
HALF = 32
NSUB = 16

f32 = jnp.float32


@functools.cache
def _mesh():
    return plsc.VectorSubcoreMesh(core_axis_name="c", subcore_axis_name="s")


_SC_PARAMS = pltpu.CompilerParams(use_tc_tiling_on_sc=False)


def _mlp2_body(x_ref, w1_ref, b1_ref, w2_ref, b2_ref, o_ref, *, final_relu):
    y = jnp.dot(x_ref[...], w1_ref[...], preferred_element_type=f32) + b1_ref[...]
    y = jnp.maximum(y, 0.0)
    z = jnp.dot(y, w2_ref[...], preferred_element_type=f32) + b2_ref[...]
    if final_relu:
        z = jnp.maximum(z, 0.0)
    o_ref[...] = z


def _mlp2(x, w1, b1, w2, b2, blk, final_relu=True):
    n, din = x.shape
    dout = w2.shape[1]
    return pl.pallas_call(
        functools.partial(_mlp2_body, final_relu=final_relu),
        grid=(n // blk,),
        in_specs=[
            pl.BlockSpec((blk, din), lambda i: (i, 0)),
            pl.BlockSpec(w1.shape, lambda i: (0, 0)),
            pl.BlockSpec(b1.shape, lambda i: (0, 0)),
            pl.BlockSpec(w2.shape, lambda i: (0, 0)),
            pl.BlockSpec(b2.shape, lambda i: (0, 0)),
        ],
        out_specs=pl.BlockSpec((blk, dout), lambda i: (i, 0)),
        out_shape=jax.ShapeDtypeStruct((n, dout), f32),
    )(x, w1, b1, w2, b2)


def _msg_body(e_ref, hs_ref, hr_ref, w1e, w1s, w1r, b1, w2, b2, lo_ref, hi_ref):
    y = (jnp.dot(e_ref[...], w1e[...], preferred_element_type=f32)
         + jnp.dot(hs_ref[...], w1s[...], preferred_element_type=f32)
         + jnp.dot(hr_ref[...], w1r[...], preferred_element_type=f32)
         + b1[...])
    y = jnp.maximum(y, 0.0)
    z = jnp.dot(y, w2[...], preferred_element_type=f32) + b2[...]
    z = jnp.maximum(z, 0.0)
    lo_ref[...] = z[:, :HALF]
    hi_ref[...] = z[:, HALF:]


def _msg(e, hs, hr, w1e, w1s, w1r, b1, w2, b2):
    wspec = lambda w: pl.BlockSpec(w.shape, lambda i: (0, 0))
    return pl.pallas_call(
        _msg_body,
        grid=(PE // EDGE_BLK,),
        in_specs=[
            pl.BlockSpec((EDGE_BLK, D), lambda i: (i, 0)),
            pl.BlockSpec((EDGE_BLK, D), lambda i: (i, 0)),
            pl.BlockSpec((EDGE_BLK, D), lambda i: (i, 0)),
            wspec(w1e), wspec(w1s), wspec(w1r), wspec(b1), wspec(w2), wspec(b2),
        ],
        out_specs=[
            pl.BlockSpec((EDGE_BLK, HALF), lambda i: (i, 0)),
            pl.BlockSpec((EDGE_BLK, HALF), lambda i: (i, 0)),
        ],
        out_shape=[
            jax.ShapeDtypeStruct((PE, HALF), f32),
            jax.ShapeDtypeStruct((PE, HALF), f32),
        ],
    )(e, hs, hr, w1e, w1s, w1r, b1, w2, b2)


def _upd_body(h_ref, al_ref, ah_ref, u1h, u1a, u1b, b1, u2, b2, o_ref):
    y = (jnp.dot(h_ref[...], u1h[...], preferred_element_type=f32)
         + jnp.dot(al_ref[...], u1a[...], preferred_element_type=f32)
         + jnp.dot(ah_ref[...], u1b[...], preferred_element_type=f32)
         + b1[...])
    y = jnp.maximum(y, 0.0)
    z = jnp.dot(y, u2[...], preferred_element_type=f32) + b2[...]
    z = jnp.maximum(z, 0.0)
    o_ref[...] = h_ref[...] + z


def _upd(h, agg_lo, agg_hi, u1h, u1a, u1b, b1, u2, b2):
    wspec = lambda w: pl.BlockSpec(w.shape, lambda i: (0, 0))
    return pl.pallas_call(
        _upd_body,
        grid=(PN // NODE_BLK,),
        in_specs=[
            pl.BlockSpec((NODE_BLK, D), lambda i: (i, 0)),
            pl.BlockSpec((NODE_BLK, HALF), lambda i: (i, 0)),
            pl.BlockSpec((NODE_BLK, HALF), lambda i: (i, 0)),
            wspec(u1h), wspec(u1a), wspec(u1b), wspec(b1), wspec(u2), wspec(b2),
        ],
        out_specs=pl.BlockSpec((NODE_BLK, D), lambda i: (i, 0)),
        out_shape=jax.ShapeDtypeStruct((PN, D), f32),
    )(h, agg_lo, agg_hi, u1h, u1a, u1b, b1, u2, b2)


@functools.cache
def _gather_kernel():
    @functools.partial(
        pl.kernel,
        mesh=_mesh(),
        out_type=(jax.ShapeDtypeStruct((PE, D), f32),
                  jax.ShapeDtypeStruct((PE, D), f32)),
        compiler_params=_SC_PARAMS,
    )
    def gather(h_hbm, s_hbm, r_hbm, hs_hbm, hr_hbm):
        def body(si_v, ri_v, os_v, or_v):
            pltpu.sync_copy(h_hbm.at[si_v.at[0]], os_v)
            pltpu.sync_copy(h_hbm.at[ri_v.at[0]], or_v)

        pltpu.emit_pipeline(
            body,
            grid=(PE // GW,),
            in_specs=[
                pl.BlockSpec((1, GW), lambda i: (0, i)),
                pl.BlockSpec((1, GW), lambda i: (0, i)),
            ],
            out_specs=[
                pl.BlockSpec((GW, D), lambda i: (i, 0)),
                pl.BlockSpec((GW, D), lambda i: (i, 0)),
            ],
            core_axis_name=("c", "s"),
            dimension_semantics=(pltpu.PARALLEL,),
        )(s_hbm, r_hbm, hs_hbm, hr_hbm)

    return gather


def _sc_gather(h, snd, rcv):
    return _gather_kernel()(h, snd, rcv)


_ROWS_PER_SUB = PN // NSUB


@functools.cache
def _scatter_kernel():
    @functools.partial(
        pl.kernel,
        mesh=_mesh(),
        out_type=(jax.ShapeDtypeStruct((PN, HALF), f32),
                  jax.ShapeDtypeStruct((PN, HALF), f32)),
        scratch_types=[
            pltpu.VMEM((GW, HALF), f32),
            pltpu.VMEM_SHARED((PN, HALF), f32),
        ],
        compiler_params=_SC_PARAMS,
    )
    def scatter(lo_hbm, hi_hbm, r_hbm, agglo_hbm, agghi_hbm, zbuf, table):
        cid = lax.axis_index("c")
        sid = lax.axis_index("s")

        zero = jnp.zeros((16,), f32)

        @pl.loop(0, GW)
        def _(r):
            @pl.loop(0, HALF, step=16)
            def _(c):
                zbuf[r, pl.ds(c, 16)] = zero

        @pl.loop(0, _ROWS_PER_SUB, step=GW)
        def _(r0):
            pltpu.sync_copy(zbuf, table.at[pl.ds(sid * _ROWS_PER_SUB + r0, GW)])

        plsc.subcore_barrier()

        def body(m_v, i_v):
            pltpu.sync_copy(m_v, table.at[i_v.at[0]], add=True)

        def run(m_hbm):
            pltpu.emit_pipeline(
                body,
                grid=(PE // GW,),
                in_specs=[
                    pl.BlockSpec((GW, HALF), lambda i: (i, 0)),
                    pl.BlockSpec((1, GW), lambda i: (0, i)),
                ],
                out_specs=[],
                core_axis_name="s",
                dimension_semantics=(pltpu.PARALLEL,),
            )(m_hbm, r_hbm)

        @pl.when(cid == 0)
        def _():
            run(lo_hbm)

        @pl.when(cid == 1)
        def _():
            run(hi_hbm)

        plsc.subcore_barrier()

        @pl.when(cid == 0)
        def _():
            pltpu.sync_copy(
                table.at[pl.ds(sid * _ROWS_PER_SUB, _ROWS_PER_SUB)],
                agglo_hbm.at[pl.ds(sid * _ROWS_PER_SUB, _ROWS_PER_SUB)])

        @pl.when(cid == 1)
        def _():
            pltpu.sync_copy(
                table.at[pl.ds(sid * _ROWS_PER_SUB, _ROWS_PER_SUB)],
                agghi_hbm.at[pl.ds(sid * _ROWS_PER_SUB, _ROWS_PER_SUB)])

    return scatter


def _sc_scatter(m_lo, m_hi, rcv):
    return _scatter_kernel()(m_lo, m_hi, rcv)


def kernel(node_in, edge_in, senders, receivers, params):
    node_p = jnp.pad(node_in, ((0, PN - N_NODES), (0, 0)))
    edge_p = jnp.pad(edge_in, ((0, PE - N_EDGES), (0, 0)))
    snd = jnp.pad(senders, (0, PE - N_EDGES),
                  constant_values=N_NODES).reshape(1, PE)
    rcv = jnp.pad(receivers, (0, PE - N_EDGES),
                  constant_values=N_NODES).reshape(1, PE)

    def b2d(b):
        return b.reshape(1, -1)

    (wn1, bn1), (wn2, bn2) = params["node_enc"]
    h = _mlp2(node_p, wn1, b2d(bn1), wn2, b2d(bn2), NODE_BLK)
    (we1, be1), (we2, be2) = params["edge_enc"]
    e = _mlp2(edge_p, we1, b2d(be1), we2, b2d(be2), EDGE_BLK)

    for k in range(STEPS):
        (w1, b1), (w2, b2) = params["msg"][k]
        hs, hr = _sc_gather(h, snd, rcv)
        m_lo, m_hi = _msg(e, hs, hr,
                          w1[:D], w1[D:2 * D], w1[2 * D:],
                          b2d(b1), w2, b2d(b2))
        agg_lo, agg_hi = _sc_scatter(m_lo, m_hi, rcv)
        (u1, c1), (u2, c2) = params["upd"][k]
        h = _upd(h, agg_lo, agg_hi,
                 u1[:D], u1[D:D + HALF], u1[D + HALF:],
                 b2d(c1), u2, b2d(c2))

    (wd1, bd1), (wd2, bd2) = params["dec"]
    out = _mlp2(h, wd1, b2d(bd1), wd2, b2d(bd2), NODE_BLK, final_relu=False)
    return out[:N_NODES]
